# baseline (speedup 1.0000x reference)
import functools

import jax
import jax.numpy as jnp
from jax import lax
from jax.experimental import pallas as pl
from jax.experimental.pallas import tpu as pltpu
from jax.experimental.pallas import tpu_sc as plsc

_N = 10000
_D = 128
_H = 256
_NP = 10112
_CH = 128
_IC = 40
_NSC = 2
_NTILE = 16
_RPT = _NP // _NTILE
_BLK = 2528


def _sc_mesh():
    return plsc.VectorSubcoreMesh(core_axis_name="c", subcore_axis_name="s")


def _agg_loop(tab_hbm, src_hbm, dst_hbm, acc_sh, src_v, dst_v, rows_v,
              sem_g, sem_s, tile_row0, n_rows_tile, src_plane):
    def gwait(b):
        pltpu.make_async_copy(tab_hbm.at[src_v.at[0]], rows_v.at[b],
                              sem_g.at[b]).wait()

    def swait(b):
        pltpu.make_async_copy(tab_hbm.at[src_v.at[0]], rows_v.at[b],
                              sem_s.at[b]).wait()

    def outer(k, carry):
        base = tile_row0 + k * _IC
        pltpu.sync_copy(src_hbm.at[pl.ds(base, _IC)], src_v)
        pltpu.sync_copy(dst_hbm.at[pl.ds(base, _IC)], dst_v)
        if src_plane is not None:
            def xform(i, carry3):
                row = src_v.at[lax.div(i, 8)]
                sl = pl.ds(lax.rem(i, 8) * 16, 16)
                row[sl] = row[sl] * 2 + src_plane
                return carry3

            lax.fori_loop(0, _IC * 8, xform, 0)
        pltpu.async_copy(tab_hbm.at[src_v.at[0]], rows_v.at[0], sem_g.at[0])

        def inner(j, carry2):
            p = lax.rem(j, 2)
            q = 1 - p

            @pl.when(j >= 1)
            def _():
                swait(q)

            @pl.when(j + 1 < _IC)
            def _():
                pltpu.async_copy(tab_hbm.at[src_v.at[j + 1]], rows_v.at[q],
                                 sem_g.at[q])

            gwait(p)
            pltpu.async_copy(rows_v.at[p], acc_sh.at[dst_v.at[j]],
                             sem_s.at[p], add=True)
            return carry2

        lax.fori_loop(0, _IC, inner, 0)
        swait((_IC - 1) % 2)
        return carry

    lax.fori_loop(0, n_rows_tile // _IC, outer, 0)


def _make_agg1(n_rows_tile):
    @functools.partial(
        pl.kernel,
        out_type=jax.ShapeDtypeStruct((_NSC, _NP, _D), jnp.float32),
        mesh=_sc_mesh(),
        scratch_types=(
            pltpu.VMEM((_IC, _CH), jnp.int32),
            pltpu.VMEM((_IC, _CH), jnp.int32),
            pltpu.VMEM((2, _CH, _D), jnp.float32),
            pltpu.VMEM_SHARED((_NP, _D), jnp.float32),
            pltpu.SemaphoreType.DMA((2,)),
            pltpu.SemaphoreType.DMA((2,)),
        ),
    )
    def agg1(x_hbm, src_hbm, dst_hbm, zeros_hbm, out_hbm,
             src_v, dst_v, rows_v, acc_sh, sem_g, sem_s):
        c = lax.axis_index("c")
        s = lax.axis_index("s")
        t = c * _NTILE + s
        r0 = s * _RPT
        pltpu.sync_copy(zeros_hbm.at[pl.ds(r0, _RPT)], acc_sh.at[pl.ds(r0, _RPT)])
        plsc.subcore_barrier()
        _agg_loop(x_hbm, src_hbm, dst_hbm, acc_sh, src_v, dst_v, rows_v,
                  sem_g, sem_s, t * n_rows_tile, n_rows_tile, None)
        plsc.subcore_barrier()
        pltpu.sync_copy(acc_sh.at[pl.ds(r0, _RPT)], out_hbm.at[c, pl.ds(r0, _RPT)])

    return agg1


def _make_agg2(n_rows_tile):
    @functools.partial(
        pl.kernel,
        out_type=jax.ShapeDtypeStruct((_NSC, _NP, _D), jnp.float32),
        mesh=_sc_mesh(),
        scratch_types=(
            pltpu.VMEM((_IC, _CH), jnp.int32),
            pltpu.VMEM((_IC, _CH), jnp.int32),
            pltpu.VMEM((2, _CH, _D), jnp.float32),
            pltpu.VMEM_SHARED((_NP, _D), jnp.float32),
            pltpu.SemaphoreType.DMA((2,)),
            pltpu.SemaphoreType.DMA((2,)),
        ),
    )
    def agg2(h2_hbm, src2_hbm, dst_hbm, zeros_hbm, out_hbm,
             src_v, dst_v, rows_v, acc_sh, sem_g, sem_s):
        c = lax.axis_index("c")
        s = lax.axis_index("s")
        r0 = s * _RPT
        pltpu.sync_copy(zeros_hbm.at[pl.ds(r0, _RPT)], acc_sh.at[pl.ds(r0, _RPT)])
        plsc.subcore_barrier()
        _agg_loop(h2_hbm, src2_hbm, dst_hbm, acc_sh, src_v, dst_v, rows_v,
                  sem_g, sem_s, s * n_rows_tile, n_rows_tile, c)
        plsc.subcore_barrier()
        pltpu.sync_copy(acc_sh.at[pl.ds(r0, _RPT)], out_hbm.at[c, pl.ds(r0, _RPT)])

    return agg2


def _make_cnt(n_rows_tile):
    @functools.partial(
        pl.kernel,
        out_type=jax.ShapeDtypeStruct((_NSC, _NP, _D), jnp.float32),
        mesh=_sc_mesh(),
        scratch_types=(
            pltpu.VMEM((_IC, _CH), jnp.int32),
            pltpu.VMEM((_CH, _D), jnp.float32),
            pltpu.VMEM_SHARED((_NP, _D), jnp.float32),
        ),
    )
    def cntk(dst_hbm, z16_hbm, ones_hbm, cnt_hbm, dst_v, ones_v, cnt_sh):
        c = lax.axis_index("c")
        s = lax.axis_index("s")
        t = c * _NTILE + s
        r0 = s * _RPT
        pltpu.sync_copy(z16_hbm.at[pl.ds(r0, _RPT)], cnt_sh.at[pl.ds(r0, _RPT)])
        pltpu.sync_copy(ones_hbm, ones_v)
        plsc.subcore_barrier()

        def outer(k, carry):
            base = t * n_rows_tile + k * _IC
            pltpu.sync_copy(dst_hbm.at[pl.ds(base, _IC)], dst_v)

            def inner(j, carry2):
                pltpu.sync_copy(ones_v, cnt_sh.at[dst_v.at[j]], add=True)
                return carry2

            lax.fori_loop(0, _IC, inner, 0)
            return carry

        lax.fori_loop(0, n_rows_tile // _IC, outer, 0)
        plsc.subcore_barrier()
        pltpu.sync_copy(cnt_sh.at[pl.ds(r0, _RPT)], cnt_hbm.at[c, pl.ds(r0, _RPT)])

    return cntk


def _tcr_body(x_ref, w_ref, b_ref, out_ref):
    o = lax.dot_general(x_ref[...], w_ref[...], (((1,), (1,)), ((), ())),
                        preferred_element_type=jnp.float32)
    out_ref[...] = o + b_ref[...]


def _tc1_body(s1_ref, cnt_ref, hr_ref, w1l_ref, h_ref):
    cnt = cnt_ref[0, :, 0:1] + cnt_ref[1, :, 0:1]
    inv = 1.0 / jnp.maximum(cnt, 1.0)
    agg = (s1_ref[0] + s1_ref[1]) * inv
    h = lax.dot_general(agg, w1l_ref[...], (((1,), (1,)), ((), ())),
                        preferred_element_type=jnp.float32)
    h_ref[...] = jnp.maximum(h + hr_ref[...], 0.0)


def _tc2_body(s2_ref, cnt_ref, or_ref, w2la_ref, w2lb_ref, out_ref):
    cnt = cnt_ref[0, :, 0:1] + cnt_ref[1, :, 0:1]
    inv = 1.0 / jnp.maximum(cnt, 1.0)
    o = lax.dot_general(s2_ref[0] * inv, w2la_ref[...], (((1,), (1,)), ((), ())),
                        preferred_element_type=jnp.float32)
    o += lax.dot_general(s2_ref[1] * inv, w2lb_ref[...], (((1,), (1,)), ((), ())),
                         preferred_element_type=jnp.float32)
    out_ref[...] = o + or_ref[...]


def _tcr(x, W, b_2d, width):
    return pl.pallas_call(
        _tcr_body,
        grid=(_NP // _BLK,),
        in_specs=[
            pl.BlockSpec((_BLK, width), lambda i: (i, 0)),
            pl.BlockSpec((_H, width), lambda i: (0, 0)),
            pl.BlockSpec((1, _H), lambda i: (0, 0)),
        ],
        out_specs=pl.BlockSpec((_BLK, _H), lambda i: (i, 0)),
        out_shape=jax.ShapeDtypeStruct((_NP, _H), jnp.float32),
    )(x, W, b_2d)


def _tc1(s1, cnt, hr, W1l):
    return pl.pallas_call(
        _tc1_body,
        grid=(_NP // _BLK,),
        in_specs=[
            pl.BlockSpec((_NSC, _BLK, _D), lambda i: (0, i, 0)),
            pl.BlockSpec((_NSC, _BLK, _D), lambda i: (0, i, 0)),
            pl.BlockSpec((_BLK, _H), lambda i: (i, 0)),
            pl.BlockSpec((_H, _D), lambda i: (0, 0)),
        ],
        out_specs=pl.BlockSpec((_BLK, _H), lambda i: (i, 0)),
        out_shape=jax.ShapeDtypeStruct((_NP, _H), jnp.float32),
    )(s1, cnt, hr, W1l)


def _tc2(s2, cnt, outr, W2la, W2lb):
    return pl.pallas_call(
        _tc2_body,
        grid=(_NP // _BLK,),
        in_specs=[
            pl.BlockSpec((_NSC, _BLK, _D), lambda i: (0, i, 0)),
            pl.BlockSpec((_NSC, _BLK, _D), lambda i: (0, i, 0)),
            pl.BlockSpec((_BLK, _H), lambda i: (i, 0)),
            pl.BlockSpec((_H, _D), lambda i: (0, 0)),
            pl.BlockSpec((_H, _D), lambda i: (0, 0)),
        ],
        out_specs=pl.BlockSpec((_BLK, _H), lambda i: (i, 0)),
        out_shape=jax.ShapeDtypeStruct((_NP, _H), jnp.float32),
    )(s2, cnt, outr, W2la, W2lb)


def kernel(x, edge_index, W1l, b1, W1r, W2l, b2, W2r):
    src = edge_index[0]
    dst = edge_index[1]
    e = src.shape[0]
    chunk_all = _NSC * _NTILE * _CH * 8
    e_pad = ((e + chunk_all - 1) // chunk_all) * chunk_all
    pad = e_pad - e
    fill = jnp.arange(pad, dtype=jnp.int32)
    src_p = jnp.concatenate([src, fill % _N])
    dst_p = jnp.concatenate([dst, _N + fill % (_NP - _N)])
    rows = e_pad // _CH
    src2d = src_p.reshape(rows, _CH)
    dst2d = dst_p.reshape(rows, _CH)
    xpad = jnp.concatenate([x, jnp.zeros((_NP - _N, _D), x.dtype)])
    z128 = jnp.zeros((_NP, _D), jnp.float32)
    ones = jnp.ones((_CH, _D), jnp.float32)

    n_rt1 = rows // (_NSC * _NTILE)
    n_rt2 = rows // _NTILE

    s1 = _make_agg1(n_rt1)(xpad, src2d, dst2d, z128)
    cnt = _make_cnt(n_rt1)(dst2d, z128, ones)
    hr = _tcr(xpad, W1r, b1.reshape(1, _H), _D)

    h = _tc1(s1, cnt, hr, W1l)

    h2 = h.reshape(_NP * 2, _D)
    s2 = _make_agg2(n_rt2)(h2, src2d, dst2d, z128)
    outr = _tcr(h, W2r, b2.reshape(1, _H), _H)

    out = _tc2(s2, cnt, outr, W2l[:, :_D], W2l[:, _D:])
    return out[:_N]


# speedup vs baseline: 1.0214x; 1.0214x over previous
import functools

import jax
import jax.numpy as jnp
from jax import lax
from jax.experimental import pallas as pl
from jax.experimental.pallas import tpu as pltpu
from jax.experimental.pallas import tpu_sc as plsc

_N = 10000
_D = 128
_H = 256
_NP = 10112
_CH = 128
_IC = 40
_NSC = 2
_NTILE = 16
_RPT = _NP // _NTILE
_BLK = 2000


def _sc_mesh():
    return plsc.VectorSubcoreMesh(core_axis_name="c", subcore_axis_name="s")


def _agg_loop(tab_hbm, src_hbm, dst_hbm, acc_sh, src_v, dst_v, rows_v,
              sem_g, sem_s, tile_row0, n_rows_tile, src_plane):
    def gwait(b):
        pltpu.make_async_copy(tab_hbm.at[src_v.at[0]], rows_v.at[b],
                              sem_g.at[b]).wait()

    def swait(b):
        pltpu.make_async_copy(tab_hbm.at[src_v.at[0]], rows_v.at[b],
                              sem_s.at[b]).wait()

    def outer(k, carry):
        base = tile_row0 + k * _IC
        pltpu.sync_copy(src_hbm.at[pl.ds(base, _IC)], src_v)
        pltpu.sync_copy(dst_hbm.at[pl.ds(base, _IC)], dst_v)
        if src_plane is not None:
            def xform(i, carry3):
                row = src_v.at[lax.div(i, 8)]
                sl = pl.ds(lax.rem(i, 8) * 16, 16)
                row[sl] = row[sl] * 2 + src_plane
                return carry3

            lax.fori_loop(0, _IC * 8, xform, 0)
        pltpu.async_copy(tab_hbm.at[src_v.at[0]], rows_v.at[0], sem_g.at[0])

        def inner(j, carry2):
            p = lax.rem(j, 2)
            q = 1 - p

            @pl.when(j >= 1)
            def _():
                swait(q)

            @pl.when(j + 1 < _IC)
            def _():
                pltpu.async_copy(tab_hbm.at[src_v.at[j + 1]], rows_v.at[q],
                                 sem_g.at[q])

            gwait(p)
            pltpu.async_copy(rows_v.at[p], acc_sh.at[dst_v.at[j]],
                             sem_s.at[p], add=True)
            return carry2

        lax.fori_loop(0, _IC, inner, 0)
        swait((_IC - 1) % 2)
        return carry

    lax.fori_loop(0, n_rows_tile // _IC, outer, 0)


def _make_agg1(n_rows_tile):
    @functools.partial(
        pl.kernel,
        out_type=jax.ShapeDtypeStruct((_NSC, _NP, _D), jnp.float32),
        mesh=_sc_mesh(),
        scratch_types=(
            pltpu.VMEM((_IC, _CH), jnp.int32),
            pltpu.VMEM((_IC, _CH), jnp.int32),
            pltpu.VMEM((2, _CH, _D), jnp.float32),
            pltpu.VMEM_SHARED((_NP, _D), jnp.float32),
            pltpu.SemaphoreType.DMA((2,)),
            pltpu.SemaphoreType.DMA((2,)),
        ),
    )
    def agg1(x_hbm, src_hbm, dst_hbm, zeros_hbm, out_hbm,
             src_v, dst_v, rows_v, acc_sh, sem_g, sem_s):
        c = lax.axis_index("c")
        s = lax.axis_index("s")
        t = c * _NTILE + s
        r0 = s * _RPT
        pltpu.sync_copy(zeros_hbm.at[pl.ds(r0, _RPT)], acc_sh.at[pl.ds(r0, _RPT)])
        plsc.subcore_barrier()
        _agg_loop(x_hbm, src_hbm, dst_hbm, acc_sh, src_v, dst_v, rows_v,
                  sem_g, sem_s, t * n_rows_tile, n_rows_tile, None)
        plsc.subcore_barrier()
        pltpu.sync_copy(acc_sh.at[pl.ds(r0, _RPT)], out_hbm.at[c, pl.ds(r0, _RPT)])

    return agg1


def _make_agg2(n_rows_tile):
    @functools.partial(
        pl.kernel,
        out_type=jax.ShapeDtypeStruct((_NSC, _NP, _D), jnp.float32),
        mesh=_sc_mesh(),
        scratch_types=(
            pltpu.VMEM((_IC, _CH), jnp.int32),
            pltpu.VMEM((_IC, _CH), jnp.int32),
            pltpu.VMEM((2, _CH, _D), jnp.float32),
            pltpu.VMEM_SHARED((_NP, _D), jnp.float32),
            pltpu.SemaphoreType.DMA((2,)),
            pltpu.SemaphoreType.DMA((2,)),
        ),
    )
    def agg2(h2_hbm, src2_hbm, dst_hbm, zeros_hbm, out_hbm,
             src_v, dst_v, rows_v, acc_sh, sem_g, sem_s):
        c = lax.axis_index("c")
        s = lax.axis_index("s")
        r0 = s * _RPT
        pltpu.sync_copy(zeros_hbm.at[pl.ds(r0, _RPT)], acc_sh.at[pl.ds(r0, _RPT)])
        plsc.subcore_barrier()
        _agg_loop(h2_hbm, src2_hbm, dst_hbm, acc_sh, src_v, dst_v, rows_v,
                  sem_g, sem_s, s * n_rows_tile, n_rows_tile, c)
        plsc.subcore_barrier()
        pltpu.sync_copy(acc_sh.at[pl.ds(r0, _RPT)], out_hbm.at[c, pl.ds(r0, _RPT)])

    return agg2


def _make_cnt(n_rows_tile):
    @functools.partial(
        pl.kernel,
        out_type=jax.ShapeDtypeStruct((_NSC, _NP, _D), jnp.float32),
        mesh=_sc_mesh(),
        scratch_types=(
            pltpu.VMEM((_IC, _CH), jnp.int32),
            pltpu.VMEM((_CH, _D), jnp.float32),
            pltpu.VMEM_SHARED((_NP, _D), jnp.float32),
        ),
    )
    def cntk(dst_hbm, z16_hbm, ones_hbm, cnt_hbm, dst_v, ones_v, cnt_sh):
        c = lax.axis_index("c")
        s = lax.axis_index("s")
        t = c * _NTILE + s
        r0 = s * _RPT
        pltpu.sync_copy(z16_hbm.at[pl.ds(r0, _RPT)], cnt_sh.at[pl.ds(r0, _RPT)])
        pltpu.sync_copy(ones_hbm, ones_v)
        plsc.subcore_barrier()

        def outer(k, carry):
            base = t * n_rows_tile + k * _IC
            pltpu.sync_copy(dst_hbm.at[pl.ds(base, _IC)], dst_v)

            def inner(j, carry2):
                pltpu.sync_copy(ones_v, cnt_sh.at[dst_v.at[j]], add=True)
                return carry2

            lax.fori_loop(0, _IC, inner, 0)
            return carry

        lax.fori_loop(0, n_rows_tile // _IC, outer, 0)
        plsc.subcore_barrier()
        pltpu.sync_copy(cnt_sh.at[pl.ds(r0, _RPT)], cnt_hbm.at[c, pl.ds(r0, _RPT)])

    return cntk


def _tcr_body(x_ref, w_ref, b_ref, out_ref):
    o = lax.dot_general(x_ref[...], w_ref[...], (((1,), (1,)), ((), ())),
                        preferred_element_type=jnp.float32)
    out_ref[...] = o + b_ref[...]


def _tc1_body(s1_ref, cnt_ref, hr_ref, w1l_ref, h_ref):
    cnt = cnt_ref[0, :, 0:1] + cnt_ref[1, :, 0:1]
    inv = 1.0 / jnp.maximum(cnt, 1.0)
    agg = (s1_ref[0] + s1_ref[1]) * inv
    h = lax.dot_general(agg, w1l_ref[...], (((1,), (1,)), ((), ())),
                        preferred_element_type=jnp.float32)
    h_ref[...] = jnp.maximum(h + hr_ref[...], 0.0)


def _tc2_body(s2_ref, cnt_ref, or_ref, w2la_ref, w2lb_ref, out_ref):
    cnt = cnt_ref[0, :, 0:1] + cnt_ref[1, :, 0:1]
    inv = 1.0 / jnp.maximum(cnt, 1.0)
    o = lax.dot_general(s2_ref[0] * inv, w2la_ref[...], (((1,), (1,)), ((), ())),
                        preferred_element_type=jnp.float32)
    o += lax.dot_general(s2_ref[1] * inv, w2lb_ref[...], (((1,), (1,)), ((), ())),
                         preferred_element_type=jnp.float32)
    out_ref[...] = o + or_ref[...]


def _tcr(x, W, b_2d, width):
    return pl.pallas_call(
        _tcr_body,
        grid=(_N // _BLK,),
        in_specs=[
            pl.BlockSpec((_BLK, width), lambda i: (i, 0)),
            pl.BlockSpec((_H, width), lambda i: (0, 0)),
            pl.BlockSpec((1, _H), lambda i: (0, 0)),
        ],
        out_specs=pl.BlockSpec((_BLK, _H), lambda i: (i, 0)),
        out_shape=jax.ShapeDtypeStruct((_N, _H), jnp.float32),
    )(x, W, b_2d)


def _tc1(s1, cnt, hr, W1l):
    return pl.pallas_call(
        _tc1_body,
        grid=(_N // _BLK,),
        in_specs=[
            pl.BlockSpec((_NSC, _BLK, _D), lambda i: (0, i, 0)),
            pl.BlockSpec((_NSC, _BLK, _D), lambda i: (0, i, 0)),
            pl.BlockSpec((_BLK, _H), lambda i: (i, 0)),
            pl.BlockSpec((_H, _D), lambda i: (0, 0)),
        ],
        out_specs=pl.BlockSpec((_BLK, _H), lambda i: (i, 0)),
        out_shape=jax.ShapeDtypeStruct((_N, _H), jnp.float32),
    )(s1, cnt, hr, W1l)


def _tc2(s2, cnt, outr, W2la, W2lb):
    return pl.pallas_call(
        _tc2_body,
        grid=(_N // _BLK,),
        in_specs=[
            pl.BlockSpec((_NSC, _BLK, _D), lambda i: (0, i, 0)),
            pl.BlockSpec((_NSC, _BLK, _D), lambda i: (0, i, 0)),
            pl.BlockSpec((_BLK, _H), lambda i: (i, 0)),
            pl.BlockSpec((_H, _D), lambda i: (0, 0)),
            pl.BlockSpec((_H, _D), lambda i: (0, 0)),
        ],
        out_specs=pl.BlockSpec((_BLK, _H), lambda i: (i, 0)),
        out_shape=jax.ShapeDtypeStruct((_N, _H), jnp.float32),
    )(s2, cnt, outr, W2la, W2lb)


def kernel(x, edge_index, W1l, b1, W1r, W2l, b2, W2r):
    src = edge_index[0]
    dst = edge_index[1]
    e = src.shape[0]
    chunk_all = _NSC * _NTILE * _CH * 8
    e_pad = ((e + chunk_all - 1) // chunk_all) * chunk_all
    pad = e_pad - e
    fill = jnp.arange(pad, dtype=jnp.int32)
    src_p = jnp.concatenate([src, fill % _N])
    dst_p = jnp.concatenate([dst, _N + fill % (_NP - _N)])
    rows = e_pad // _CH
    src2d = src_p.reshape(rows, _CH)
    dst2d = dst_p.reshape(rows, _CH)
    z128 = jnp.zeros((_NP, _D), jnp.float32)
    ones = jnp.ones((_CH, _D), jnp.float32)

    n_rt1 = rows // (_NSC * _NTILE)
    n_rt2 = rows // _NTILE

    s1 = _make_agg1(n_rt1)(x, src2d, dst2d, z128)
    cnt = _make_cnt(n_rt1)(dst2d, z128, ones)
    hr = _tcr(x, W1r, b1.reshape(1, _H), _D)

    h = _tc1(s1, cnt, hr, W1l)

    h2 = h.reshape(_N * 2, _D)
    s2 = _make_agg2(n_rt2)(h2, src2d, dst2d, z128)
    outr = _tcr(h, W2r, b2.reshape(1, _H), _H)

    return _tc2(s2, cnt, outr, W2l[:, :_D], W2l[:, _D:])
